# Initial kernel scaffold; baseline (speedup 1.0000x reference)
import jax
import jax.numpy as jnp
from jax.experimental import pallas as pl


def kernel(x, edge_index, W1, att_src1, att_dst1, b1, W2, att_src2, att_dst2, b2):
    raise NotImplementedError("write your pallas kernel here")


# speedup vs baseline: 8.0547x; 8.0547x over previous
import functools

import jax
import jax.numpy as jnp
from jax import lax
from jax.experimental import pallas as pl
from jax.experimental.pallas import tpu as pltpu
from jax.experimental.pallas import tpu_sc as plsc

N = 10000
NP = 10240
NBLK = 8
BLK = 1280
D_IN = 256
HEADS = 8
NC = 40
NCP = 128
PLANES = 16
PW = 128
E = 160000
ET = E + N
EP = 172032
KB = 128
CT1 = EP // 16
B1 = CT1 // KB
CT2 = EP // 2 // 16
B2 = CT2 // KB
DR = NP // 128
DRT = DR // 16
PKB = 14
PKM = (1 << PKB) - 1

_f32 = jnp.float32
_i32 = jnp.int32


def _mesh():
    return plsc.VectorSubcoreMesh(core_axis_name="c", subcore_axis_name="s",
                                  num_cores=2, num_subcores=16)


_SC_CP = functools.partial(pltpu.CompilerParams, needs_layout_passes=False)


def _tc1_body(x_ref, w_ref, ats_ref, atd_ref, h_ref, as_ref, ad_ref):
    p = pl.program_id(1)
    h = jnp.dot(x_ref[...], w_ref[...], preferred_element_type=_f32)
    h_ref[0] = h
    asp = jnp.sum(h * ats_ref[0], axis=1)
    adp = jnp.sum(h * atd_ref[0], axis=1)

    @pl.when(p % 2 == 0)
    def _():
        as_ref[0, 0, 0] = asp
        ad_ref[0, 0, 0] = adp

    @pl.when(p % 2 == 1)
    def _():
        as_ref[0, 0, 0] += asp
        ad_ref[0, 0, 0] += adp


def _tc1(x_pad, W1, ats, atd):
    return pl.pallas_call(
        _tc1_body,
        grid=(NBLK, PLANES),
        in_specs=[
            pl.BlockSpec((BLK, D_IN), lambda i, p: (i, 0)),
            pl.BlockSpec((D_IN, PW), lambda i, p: (0, p)),
            pl.BlockSpec((1, 1, PW), lambda i, p: (p, 0, 0)),
            pl.BlockSpec((1, 1, PW), lambda i, p: (p, 0, 0)),
        ],
        out_specs=[
            pl.BlockSpec((1, BLK, PW), lambda i, p: (p, i, 0)),
            pl.BlockSpec((1, 1, 1, BLK), lambda i, p: (p // 2, i, 0, 0)),
            pl.BlockSpec((1, 1, 1, BLK), lambda i, p: (p // 2, i, 0, 0)),
        ],
        out_shape=[
            jax.ShapeDtypeStruct((PLANES, NP, PW), _f32),
            jax.ShapeDtypeStruct((HEADS, NBLK, 1, BLK), _f32),
            jax.ShapeDtypeStruct((HEADS, NBLK, 1, BLK), _f32),
        ],
    )(x_pad, W1, ats, atd)


def _zero_vmem_rows(ref, nrows, width):
    z16 = jnp.zeros((16,), _f32)

    @pl.loop(0, nrows)
    def _(r):
        for j in range(width // 16):
            ref[r, pl.ds(16 * j, 16)] = z16


def _build_den_idx(den_idx):
    lane = jnp.arange(16, dtype=_i32)
    for g in range(DR // 16):
        den_idx[0, pl.ds(16 * g, 16)] = 16 * g + lane


def _edge_weights(pk_t, a_s, a_d, w_t, den_vm, ngroups):
    @pl.loop(0, ngroups)
    def _(g):
        pk16 = pk_t[pl.ds(16 * g, 16)]
        s16 = lax.shift_right_logical(pk16, PKB)
        d16 = jnp.bitwise_and(pk16, PKM)
        av = plsc.load_gather(a_s, [s16]) + plsc.load_gather(a_d, [d16])
        av = jnp.where(av >= 0.0, av, av * jnp.float32(0.2))
        w = jnp.exp(av)
        w_t[pl.ds(16 * g, 16)] = w
        row = lax.shift_right_logical(d16, 7)
        col = jnp.bitwise_and(d16, 127)
        plsc.addupdate_scatter(den_vm, [row, col], w)


def _scores_body(as_hbm, ad_hbm, pk_hbm, w_hbm, den_hbm,
                 pk_t, a_s, a_d, den_vm, w_tile, zden, den_idx, bnc, den_sh,
                 *, nheads, edges_per_tile, den_rows_out):
    cid = lax.axis_index("c")
    sid = lax.axis_index("s")

    if nheads == 1:
        e0 = cid * (EP // 2) + sid * edges_per_tile
    else:
        e0 = sid * edges_per_tile
    pltpu.sync_copy(pk_hbm.at[pl.ds(e0, edges_per_tile)], pk_t)
    _zero_vmem_rows(zden, 8, 128)
    _build_den_idx(den_idx)

    @pl.loop(0, nheads)
    def _(hh):
        head = cid * 4 + hh if nheads > 1 else jnp.int32(0)
        pltpu.sync_copy(as_hbm.at[pl.ds(head * NP, NP)], a_s)
        pltpu.sync_copy(ad_hbm.at[pl.ds(head * NP, NP)], a_d)
        _zero_vmem_rows(den_vm, DR, 128)

        @pl.when(sid < 10)
        def _():
            pltpu.sync_copy(zden, den_sh.at[pl.ds(sid * 8, 8)])

        plsc.subcore_barrier()
        _edge_weights(pk_t, a_s, a_d, w_tile, den_vm, edges_per_tile // 16)
        pltpu.sync_copy(den_vm, den_sh.at[den_idx.at[0]], add=True)
        plsc.subcore_barrier()

        @pl.when(sid < 10)
        def _():
            pltpu.sync_copy(den_sh.at[pl.ds(sid * 8, 8)], bnc)
            if nheads == 1:
                pltpu.sync_copy(
                    bnc, den_hbm.at[pl.ds(cid * DR + sid * 8, 8)])
            else:
                pltpu.sync_copy(bnc, den_hbm.at[head, pl.ds(sid * 8, 8)])

        if nheads == 1:
            pltpu.sync_copy(w_tile, w_hbm.at[pl.ds(e0, edges_per_tile)])
        else:
            pltpu.sync_copy(
                w_tile,
                w_hbm.at[pl.ds(head * EP + sid * edges_per_tile,
                               edges_per_tile)])


def _make_sc1a():
    body = functools.partial(_scores_body, nheads=4,
                             edges_per_tile=CT1, den_rows_out=DR)
    return pl.kernel(
        body,
        out_type=[
            jax.ShapeDtypeStruct((HEADS * EP,), _f32),
            jax.ShapeDtypeStruct((HEADS, DR, 128), _f32),
        ],
        mesh=_mesh(),
        compiler_params=_SC_CP(),
        scratch_types=[
            pltpu.VMEM((CT1,), _i32),
            pltpu.VMEM((NP,), _f32),
            pltpu.VMEM((NP,), _f32),
            pltpu.VMEM((DR, 128), _f32),
            pltpu.VMEM((CT1,), _f32),
            pltpu.VMEM((8, 128), _f32),
            pltpu.VMEM((1, DR), _i32),
            pltpu.VMEM((8, 128), _f32),
            pltpu.VMEM_SHARED((DR, 128), _f32),
        ],
    )


def _make_sc2a():
    body = functools.partial(_scores_body, nheads=1,
                             edges_per_tile=CT2, den_rows_out=2 * DR)
    return pl.kernel(
        body,
        out_type=[
            jax.ShapeDtypeStruct((EP,), _f32),
            jax.ShapeDtypeStruct((2 * DR, 128), _f32),
        ],
        mesh=_mesh(),
        compiler_params=_SC_CP(),
        scratch_types=[
            pltpu.VMEM((CT2,), _i32),
            pltpu.VMEM((NP,), _f32),
            pltpu.VMEM((NP,), _f32),
            pltpu.VMEM((DR, 128), _f32),
            pltpu.VMEM((CT2,), _f32),
            pltpu.VMEM((8, 128), _f32),
            pltpu.VMEM((1, DR), _i32),
            pltpu.VMEM((8, 128), _f32),
            pltpu.VMEM_SHARED((DR, 128), _f32),
        ],
    )


def _agg_loop(pk_t, w_t, idxg, dstb, rows, acc_sh, h_hbm, sem,
              pbase, nbatches, width):
    @pl.loop(0, nbatches)
    def _(b):
        base = b * KB
        for j in range(8):
            pk16 = pk_t[pl.ds(base + 16 * j, 16)]
            idxg[pl.ds(16 * j, 16)] = (
                lax.shift_right_logical(pk16, PKB) + pbase)
            dstb[pl.ds(16 * j, 16)] = jnp.bitwise_and(pk16, PKM)
        pltpu.async_copy(h_hbm.at[idxg], rows, sem).wait()

        @pl.loop(0, KB)
        def _(e):
            wv = plsc.load_gather(w_t, [jnp.full((16,), base + e, _i32)])
            for j in range(width // 16):
                rows[e, pl.ds(16 * j, 16)] = (
                    rows[e, pl.ds(16 * j, 16)] * wv)

        pltpu.sync_copy(rows, acc_sh.at[dstb], add=True)


def _sc1b_body(h1_hbm, w_hbm, pk_hbm, acc_hbm,
               pk_t, w_t, idxg, dstb, rows, zrow, acc_sh, sem):
    cid = lax.axis_index("c")
    sid = lax.axis_index("s")

    pltpu.sync_copy(pk_hbm.at[pl.ds(sid * CT1, CT1)], pk_t)
    _zero_vmem_rows(zrow, 16, PW)

    @pl.loop(0, 4)
    def _(hh):
        head = cid * 4 + hh
        pltpu.sync_copy(w_hbm.at[pl.ds(head * EP + sid * CT1, CT1)], w_t)

        for half in range(2):
            pbase = (head * 2 + half) * NP

            @pl.loop(0, 40)
            def _(k):
                pltpu.sync_copy(zrow,
                                acc_sh.at[pl.ds(sid * 640 + 16 * k, 16)])

            plsc.subcore_barrier()
            _agg_loop(pk_t, w_t, idxg, dstb, rows, acc_sh, h1_hbm, sem,
                      pbase, B1, PW)
            plsc.subcore_barrier()

            @pl.loop(0, 5)
            def _(k):
                r0 = sid * 640 + 128 * k
                pltpu.sync_copy(acc_sh.at[pl.ds(r0, 128)], rows)
                pltpu.sync_copy(rows, acc_hbm.at[pl.ds(pbase + r0, 128)])

            plsc.subcore_barrier()


def _make_sc1b():
    return pl.kernel(
        _sc1b_body,
        out_type=jax.ShapeDtypeStruct((PLANES * NP, PW), _f32),
        mesh=_mesh(),
        compiler_params=_SC_CP(),
        scratch_types=[
            pltpu.VMEM((CT1,), _i32),
            pltpu.VMEM((CT1,), _f32),
            pltpu.VMEM((KB,), _i32),
            pltpu.VMEM((KB,), _i32),
            pltpu.VMEM((KB, PW), _f32),
            pltpu.VMEM((16, PW), _f32),
            pltpu.VMEM_SHARED((NP, PW), _f32),
            pltpu.SemaphoreType.DMA,
        ],
    )


def _sc2b_body(h2_hbm, w_hbm, pk_hbm, acc_hbm,
               pk_t, w_t, idxg, dstb, rows, zrow, acc_sh, sem):
    cid = lax.axis_index("c")
    sid = lax.axis_index("s")

    e0 = cid * (EP // 2) + sid * CT2
    pltpu.sync_copy(pk_hbm.at[pl.ds(e0, CT2)], pk_t)
    pltpu.sync_copy(w_hbm.at[pl.ds(e0, CT2)], w_t)
    _zero_vmem_rows(zrow, 16, NCP)

    @pl.loop(0, 40)
    def _(k):
        pltpu.sync_copy(zrow, acc_sh.at[pl.ds(sid * 640 + 16 * k, 16)])

    plsc.subcore_barrier()
    _agg_loop(pk_t, w_t, idxg, dstb, rows, acc_sh, h2_hbm, sem,
              jnp.int32(0), B2, NCP)
    plsc.subcore_barrier()

    @pl.loop(0, 5)
    def _(k):
        r0 = sid * 640 + 128 * k
        pltpu.sync_copy(acc_sh.at[pl.ds(r0, 128)], rows)
        pltpu.sync_copy(rows, acc_hbm.at[pl.ds(cid * NP + r0, 128)])


def _make_sc2b():
    return pl.kernel(
        _sc2b_body,
        out_type=jax.ShapeDtypeStruct((2 * NP, NCP), _f32),
        mesh=_mesh(),
        compiler_params=_SC_CP(),
        scratch_types=[
            pltpu.VMEM((CT2,), _i32),
            pltpu.VMEM((CT2,), _f32),
            pltpu.VMEM((KB,), _i32),
            pltpu.VMEM((KB,), _i32),
            pltpu.VMEM((KB, NCP), _f32),
            pltpu.VMEM((16, NCP), _f32),
            pltpu.VMEM_SHARED((NP, NCP), _f32),
            pltpu.SemaphoreType.DMA,
        ],
    )


def _tc2_body(acc_ref, den_ref, b1_ref, w2_ref, a2s_ref, a2d_ref,
              h2_ref, s_ref, d_ref):
    p = pl.program_id(1)
    den = den_ref[0, 0, 0] + jnp.float32(1e-16)
    x2 = jnp.maximum(acc_ref[0] / den[:, None] + b1_ref[0], 0.0)
    hp = jnp.dot(x2, w2_ref[0], preferred_element_type=_f32)

    @pl.when(p == 0)
    def _():
        h2_ref[...] = hp

    @pl.when(p > 0)
    def _():
        h2_ref[...] += hp

    @pl.when(p == PLANES - 1)
    def _():
        h2f = h2_ref[...]
        s_ref[0, 0] = jnp.sum(h2f * a2s_ref[...], axis=1)
        d_ref[0, 0] = jnp.sum(h2f * a2d_ref[...], axis=1)


def _tc2(acc3, den4, b1r, w2r, a2s, a2d):
    return pl.pallas_call(
        _tc2_body,
        grid=(NBLK, PLANES),
        in_specs=[
            pl.BlockSpec((1, BLK, PW), lambda i, p: (p, i, 0)),
            pl.BlockSpec((1, 1, 1, BLK), lambda i, p: (p // 2, i, 0, 0)),
            pl.BlockSpec((1, 1, PW), lambda i, p: (p, 0, 0)),
            pl.BlockSpec((1, PW, NCP), lambda i, p: (p, 0, 0)),
            pl.BlockSpec((1, NCP), lambda i, p: (0, 0)),
            pl.BlockSpec((1, NCP), lambda i, p: (0, 0)),
        ],
        out_specs=[
            pl.BlockSpec((BLK, NCP), lambda i, p: (i, 0)),
            pl.BlockSpec((1, 1, BLK), lambda i, p: (i, 0, 0)),
            pl.BlockSpec((1, 1, BLK), lambda i, p: (i, 0, 0)),
        ],
        out_shape=[
            jax.ShapeDtypeStruct((NP, NCP), _f32),
            jax.ShapeDtypeStruct((NBLK, 1, BLK), _f32),
            jax.ShapeDtypeStruct((NBLK, 1, BLK), _f32),
        ],
    )(acc3, den4, b1r, w2r, a2s, a2d)


def _tc3_body(acc_ref, den_ref, b2_ref, out_ref):
    s = acc_ref[0] + acc_ref[1]
    den = den_ref[0, 0, 0] + den_ref[1, 0, 0] + jnp.float32(1e-16)
    out_ref[...] = s[:, :NC] / den[:, None] + b2_ref[...]


def _tc3(acc2, den2, b2r):
    return pl.pallas_call(
        _tc3_body,
        grid=(NBLK,),
        in_specs=[
            pl.BlockSpec((2, BLK, NCP), lambda i: (0, i, 0)),
            pl.BlockSpec((2, 1, 1, BLK), lambda i: (0, i, 0, 0)),
            pl.BlockSpec((1, NC), lambda i: (0, 0)),
        ],
        out_specs=pl.BlockSpec((BLK, NC), lambda i: (i, 0)),
        out_shape=jax.ShapeDtypeStruct((N, NC), _f32),
    )(acc2, den2, b2r)


_make_sc1a = functools.cache(_make_sc1a)
_make_sc1b = functools.cache(_make_sc1b)
_make_sc2a = functools.cache(_make_sc2a)
_make_sc2b = functools.cache(_make_sc2b)


def kernel(x, edge_index, W1, att_src1, att_dst1, b1, W2, att_src2,
           att_dst2, b2):
    x_pad = jnp.pad(x, ((0, NP - N), (0, 0)))
    loops = jnp.arange(N, dtype=edge_index.dtype)
    src = jnp.concatenate([edge_index[0], loops]).astype(_i32)
    dst = jnp.concatenate([edge_index[1], loops]).astype(_i32)
    srcp = jnp.pad(src, (0, EP - ET))
    dstp = jnp.pad(dst, (0, EP - ET), constant_values=N)
    pk = jnp.bitwise_or(jnp.left_shift(srcp, PKB), dstp)

    ats = att_src1.reshape(PLANES, 1, PW)
    atd = att_dst1.reshape(PLANES, 1, PW)
    h1g, a_sT, a_dT = _tc1(x_pad, W1, ats, atd)
    h1flat = h1g.reshape(PLANES * NP, PW)
    a_s2 = a_sT.reshape(HEADS * NP)
    a_d2 = a_dT.reshape(HEADS * NP)

    w_all, den = _make_sc1a()(a_s2, a_d2, pk)
    acc = _make_sc1b()(h1flat, w_all, pk)
    acc3 = acc.reshape(PLANES, NP, PW)
    den4 = den.reshape(HEADS, NP).reshape(HEADS, NBLK, 1, BLK)

    b1r = b1.reshape(PLANES, 1, PW)
    w2r = jnp.pad(W2, ((0, 0), (0, NCP - NC))).reshape(PLANES, PW, NCP)
    a2sp = jnp.pad(att_src2, ((0, 0), (0, NCP - NC)))
    a2dp = jnp.pad(att_dst2, ((0, 0), (0, NCP - NC)))
    h2pad, a2s, a2d = _tc2(acc3, den4, b1r, w2r, a2sp, a2dp)

    a2sr = a2s.reshape(NP)
    a2dr = a2d.reshape(NP)
    w2_all, den2 = _make_sc2a()(a2sr, a2dr, pk)
    acc2 = _make_sc2b()(h2pad, w2_all, pk)
    out = _tc3(acc2.reshape(2, NP, NCP),
               den2.reshape(2, NP).reshape(2, NBLK, 1, BLK),
               b2.reshape(1, NC))
    return out


# speedup vs baseline: 10.7017x; 1.3286x over previous
import functools

import jax
import jax.numpy as jnp
from jax import lax
from jax.experimental import pallas as pl
from jax.experimental.pallas import tpu as pltpu
from jax.experimental.pallas import tpu_sc as plsc

N = 10000
NP = 10240
NBLK = 8
BLK = 1280
D_IN = 256
HEADS = 8
NC = 40
NCP = 128
PLANES = 16
PW = 128
E = 160000
ET = E + N
EP = 172032
KB = 64
CT1 = EP // 16
B1 = CT1 // KB
CT2 = EP // 2 // 16
B2 = CT2 // KB
DR = NP // 128
DRT = DR // 16
PKB = 14
PKM = (1 << PKB) - 1

_f32 = jnp.float32
_i32 = jnp.int32


def _mesh():
    return plsc.VectorSubcoreMesh(core_axis_name="c", subcore_axis_name="s",
                                  num_cores=2, num_subcores=16)


_SC_CP = functools.partial(pltpu.CompilerParams, needs_layout_passes=False)


def _tc1_body(x_ref, w_ref, ats_ref, atd_ref, h_ref, as_ref, ad_ref):
    p = pl.program_id(1)
    h = jnp.dot(x_ref[...], w_ref[...], preferred_element_type=_f32)
    h_ref[0] = h
    asp = jnp.sum(h * ats_ref[0], axis=1)
    adp = jnp.sum(h * atd_ref[0], axis=1)

    @pl.when(p % 2 == 0)
    def _():
        as_ref[0, 0, 0] = asp
        ad_ref[0, 0, 0] = adp

    @pl.when(p % 2 == 1)
    def _():
        as_ref[0, 0, 0] += asp
        ad_ref[0, 0, 0] += adp


def _tc1(x_pad, W1, ats, atd):
    return pl.pallas_call(
        _tc1_body,
        grid=(NBLK, PLANES),
        in_specs=[
            pl.BlockSpec((BLK, D_IN), lambda i, p: (i, 0)),
            pl.BlockSpec((D_IN, PW), lambda i, p: (0, p)),
            pl.BlockSpec((1, 1, PW), lambda i, p: (p, 0, 0)),
            pl.BlockSpec((1, 1, PW), lambda i, p: (p, 0, 0)),
        ],
        out_specs=[
            pl.BlockSpec((1, BLK, PW), lambda i, p: (p, i, 0)),
            pl.BlockSpec((1, 1, 1, BLK), lambda i, p: (p // 2, i, 0, 0)),
            pl.BlockSpec((1, 1, 1, BLK), lambda i, p: (p // 2, i, 0, 0)),
        ],
        out_shape=[
            jax.ShapeDtypeStruct((PLANES, NP, PW), _f32),
            jax.ShapeDtypeStruct((HEADS, NBLK, 1, BLK), _f32),
            jax.ShapeDtypeStruct((HEADS, NBLK, 1, BLK), _f32),
        ],
    )(x_pad, W1, ats, atd)


def _zero_vmem_rows(ref, nrows, width):
    z16 = jnp.zeros((16,), _f32)

    @pl.loop(0, nrows)
    def _(r):
        for j in range(width // 16):
            ref[r, pl.ds(16 * j, 16)] = z16


def _build_den_idx(den_idx):
    lane = jnp.arange(16, dtype=_i32)
    for g in range(DR // 16):
        den_idx[0, pl.ds(16 * g, 16)] = 16 * g + lane


def _edge_weights(pk_t, a_s, a_d, w_t, den_vm, ngroups):
    @pl.loop(0, ngroups)
    def _(g):
        pk16 = pk_t[pl.ds(16 * g, 16)]
        s16 = lax.shift_right_logical(pk16, PKB)
        d16 = jnp.bitwise_and(pk16, PKM)
        av = plsc.load_gather(a_s, [s16]) + plsc.load_gather(a_d, [d16])
        av = jnp.where(av >= 0.0, av, av * jnp.float32(0.2))
        w = jnp.exp(av)
        w_t[pl.ds(16 * g, 16)] = w
        row = lax.shift_right_logical(d16, 7)
        col = jnp.bitwise_and(d16, 127)
        plsc.addupdate_scatter(den_vm, [row, col], w)


def _scores_body(as_hbm, ad_hbm, pk_hbm, w_hbm, den_hbm,
                 pk_t, a_s, a_d, den_vm, w_tile, zden, den_idx, bnc, den_sh,
                 *, nheads, edges_per_tile, den_rows_out):
    cid = lax.axis_index("c")
    sid = lax.axis_index("s")

    if nheads == 1:
        e0 = cid * (EP // 2) + sid * edges_per_tile
    else:
        e0 = sid * edges_per_tile
    pltpu.sync_copy(pk_hbm.at[pl.ds(e0, edges_per_tile)], pk_t)
    _zero_vmem_rows(zden, 8, 128)
    _build_den_idx(den_idx)

    @pl.loop(0, nheads)
    def _(hh):
        head = cid * 4 + hh if nheads > 1 else jnp.int32(0)
        pltpu.sync_copy(as_hbm.at[pl.ds(head * NP, NP)], a_s)
        pltpu.sync_copy(ad_hbm.at[pl.ds(head * NP, NP)], a_d)
        _zero_vmem_rows(den_vm, DR, 128)

        @pl.when(sid < 10)
        def _():
            pltpu.sync_copy(zden, den_sh.at[pl.ds(sid * 8, 8)])

        plsc.subcore_barrier()
        _edge_weights(pk_t, a_s, a_d, w_tile, den_vm, edges_per_tile // 16)
        pltpu.sync_copy(den_vm, den_sh.at[den_idx.at[0]], add=True)
        plsc.subcore_barrier()

        @pl.when(sid < 10)
        def _():
            pltpu.sync_copy(den_sh.at[pl.ds(sid * 8, 8)], bnc)
            if nheads == 1:
                pltpu.sync_copy(
                    bnc, den_hbm.at[pl.ds(cid * DR + sid * 8, 8)])
            else:
                pltpu.sync_copy(bnc, den_hbm.at[head, pl.ds(sid * 8, 8)])

        if nheads == 1:
            pltpu.sync_copy(w_tile, w_hbm.at[pl.ds(e0, edges_per_tile)])
        else:
            pltpu.sync_copy(
                w_tile,
                w_hbm.at[pl.ds(head * EP + sid * edges_per_tile,
                               edges_per_tile)])


def _make_sc1a():
    body = functools.partial(_scores_body, nheads=4,
                             edges_per_tile=CT1, den_rows_out=DR)
    return pl.kernel(
        body,
        out_type=[
            jax.ShapeDtypeStruct((HEADS * EP,), _f32),
            jax.ShapeDtypeStruct((HEADS, DR, 128), _f32),
        ],
        mesh=_mesh(),
        compiler_params=_SC_CP(),
        scratch_types=[
            pltpu.VMEM((CT1,), _i32),
            pltpu.VMEM((NP,), _f32),
            pltpu.VMEM((NP,), _f32),
            pltpu.VMEM((DR, 128), _f32),
            pltpu.VMEM((CT1,), _f32),
            pltpu.VMEM((8, 128), _f32),
            pltpu.VMEM((1, DR), _i32),
            pltpu.VMEM((8, 128), _f32),
            pltpu.VMEM_SHARED((DR, 128), _f32),
        ],
    )


def _make_sc2a():
    body = functools.partial(_scores_body, nheads=1,
                             edges_per_tile=CT2, den_rows_out=2 * DR)
    return pl.kernel(
        body,
        out_type=[
            jax.ShapeDtypeStruct((EP,), _f32),
            jax.ShapeDtypeStruct((2 * DR, 128), _f32),
        ],
        mesh=_mesh(),
        compiler_params=_SC_CP(),
        scratch_types=[
            pltpu.VMEM((CT2,), _i32),
            pltpu.VMEM((NP,), _f32),
            pltpu.VMEM((NP,), _f32),
            pltpu.VMEM((DR, 128), _f32),
            pltpu.VMEM((CT2,), _f32),
            pltpu.VMEM((8, 128), _f32),
            pltpu.VMEM((1, DR), _i32),
            pltpu.VMEM((8, 128), _f32),
            pltpu.VMEM_SHARED((DR, 128), _f32),
        ],
    )


def _agg_loop(pk_t, w_t, idxg, dstb, rows, acc_sh, h_hbm, sems,
              pbase, nbatches, width):
    gsem0, gsem1, ssem0, ssem1 = sems
    gsems = (gsem0, gsem1)
    ssems = (ssem0, ssem1)

    def build(pi, b):
        base = b * KB
        for j in range(KB // 16):
            pk16 = pk_t[pl.ds(base + 16 * j, 16)]
            idxg[pi, pl.ds(16 * j, 16)] = (
                lax.shift_right_logical(pk16, PKB) + pbase)
            dstb[pi, pl.ds(16 * j, 16)] = jnp.bitwise_and(pk16, PKM)

    def scale(pi, b):
        base = b * KB

        @pl.loop(0, KB)
        def _(e):
            wv = plsc.load_gather(w_t, [jnp.full((16,), base + e, _i32)])
            for j in range(width // 16):
                rows[pi, e, pl.ds(16 * j, 16)] = (
                    rows[pi, e, pl.ds(16 * j, 16)] * wv)

    build(0, jnp.int32(0))
    pltpu.async_copy(h_hbm.at[idxg.at[0]], rows.at[0], gsems[0])

    @pl.loop(0, nbatches // 2)
    def _(bb):
        for par in range(2):
            b = 2 * bb + par
            cur, oth = par, 1 - par

            @pl.when(b > 0)
            def _():
                pltpu.make_async_copy(
                    rows.at[oth], acc_sh.at[dstb.at[oth]],
                    ssems[oth]).wait()

            @pl.when(b + 1 < nbatches)
            def _():
                build(oth, b + 1)
                pltpu.async_copy(h_hbm.at[idxg.at[oth]], rows.at[oth],
                                 gsems[oth])

            pltpu.make_async_copy(h_hbm.at[idxg.at[cur]], rows.at[cur],
                                  gsems[cur]).wait()
            scale(cur, b)
            pltpu.async_copy(rows.at[cur], acc_sh.at[dstb.at[cur]],
                             ssems[cur], add=True)

    last = (nbatches - 1) % 2
    pltpu.make_async_copy(rows.at[last], acc_sh.at[dstb.at[last]],
                          ssems[last]).wait()


def _sc1b_body(h1_hbm, w_hbm, pk_hbm, acc_hbm,
               pk_t, w_t, idxg, dstb, rows, zrow, acc_sh,
               gsem0, gsem1, ssem0, ssem1):
    cid = lax.axis_index("c")
    sid = lax.axis_index("s")
    sems = (gsem0, gsem1, ssem0, ssem1)

    pltpu.sync_copy(pk_hbm.at[pl.ds(sid * CT1, CT1)], pk_t)
    _zero_vmem_rows(zrow, 16, PW)

    @pl.loop(0, 4)
    def _(hh):
        head = cid * 4 + hh
        pltpu.sync_copy(w_hbm.at[pl.ds(head * EP + sid * CT1, CT1)], w_t)

        for half in range(2):
            pbase = (head * 2 + half) * NP

            @pl.loop(0, 40)
            def _(k):
                pltpu.sync_copy(zrow,
                                acc_sh.at[pl.ds(sid * 640 + 16 * k, 16)])

            plsc.subcore_barrier()
            _agg_loop(pk_t, w_t, idxg, dstb, rows, acc_sh, h1_hbm, sems,
                      pbase, B1, PW)
            plsc.subcore_barrier()

            @pl.loop(0, 10)
            def _(k):
                r0 = sid * 640 + 64 * k
                pltpu.sync_copy(acc_sh.at[pl.ds(r0, 64)], rows.at[0])
                pltpu.sync_copy(rows.at[0],
                                acc_hbm.at[pl.ds(pbase + r0, 64)])

            plsc.subcore_barrier()


def _make_sc1b():
    return pl.kernel(
        _sc1b_body,
        out_type=jax.ShapeDtypeStruct((PLANES * NP, PW), _f32),
        mesh=_mesh(),
        compiler_params=_SC_CP(),
        scratch_types=[
            pltpu.VMEM((CT1,), _i32),
            pltpu.VMEM((CT1,), _f32),
            pltpu.VMEM((2, KB), _i32),
            pltpu.VMEM((2, KB), _i32),
            pltpu.VMEM((2, KB, PW), _f32),
            pltpu.VMEM((16, PW), _f32),
            pltpu.VMEM_SHARED((NP, PW), _f32),
            pltpu.SemaphoreType.DMA,
            pltpu.SemaphoreType.DMA,
            pltpu.SemaphoreType.DMA,
            pltpu.SemaphoreType.DMA,
        ],
    )


def _sc2b_body(h2_hbm, w_hbm, pk_hbm, acc_hbm,
               pk_t, w_t, idxg, dstb, rows, zrow, acc_sh,
               gsem0, gsem1, ssem0, ssem1):
    cid = lax.axis_index("c")
    sid = lax.axis_index("s")
    sems = (gsem0, gsem1, ssem0, ssem1)

    e0 = cid * (EP // 2) + sid * CT2
    pltpu.sync_copy(pk_hbm.at[pl.ds(e0, CT2)], pk_t)
    pltpu.sync_copy(w_hbm.at[pl.ds(e0, CT2)], w_t)
    _zero_vmem_rows(zrow, 16, NCP)

    @pl.loop(0, 40)
    def _(k):
        pltpu.sync_copy(zrow, acc_sh.at[pl.ds(sid * 640 + 16 * k, 16)])

    plsc.subcore_barrier()
    _agg_loop(pk_t, w_t, idxg, dstb, rows, acc_sh, h2_hbm, sems,
              jnp.int32(0), B2, NCP)
    plsc.subcore_barrier()

    @pl.loop(0, 10)
    def _(k):
        r0 = sid * 640 + 64 * k
        pltpu.sync_copy(acc_sh.at[pl.ds(r0, 64)], rows.at[0])
        pltpu.sync_copy(rows.at[0], acc_hbm.at[pl.ds(cid * NP + r0, 64)])


def _make_sc2b():
    return pl.kernel(
        _sc2b_body,
        out_type=jax.ShapeDtypeStruct((2 * NP, NCP), _f32),
        mesh=_mesh(),
        compiler_params=_SC_CP(),
        scratch_types=[
            pltpu.VMEM((CT2,), _i32),
            pltpu.VMEM((CT2,), _f32),
            pltpu.VMEM((2, KB), _i32),
            pltpu.VMEM((2, KB), _i32),
            pltpu.VMEM((2, KB, NCP), _f32),
            pltpu.VMEM((16, NCP), _f32),
            pltpu.VMEM_SHARED((NP, NCP), _f32),
            pltpu.SemaphoreType.DMA,
            pltpu.SemaphoreType.DMA,
            pltpu.SemaphoreType.DMA,
            pltpu.SemaphoreType.DMA,
        ],
    )


def _tc2_body(acc_ref, den_ref, b1_ref, w2_ref, a2s_ref, a2d_ref,
              h2_ref, s_ref, d_ref):
    p = pl.program_id(1)
    den = den_ref[0, 0, 0] + jnp.float32(1e-16)
    x2 = jnp.maximum(acc_ref[0] / den[:, None] + b1_ref[0], 0.0)
    hp = jnp.dot(x2, w2_ref[0], preferred_element_type=_f32)

    @pl.when(p == 0)
    def _():
        h2_ref[...] = hp

    @pl.when(p > 0)
    def _():
        h2_ref[...] += hp

    @pl.when(p == PLANES - 1)
    def _():
        h2f = h2_ref[...]
        s_ref[0, 0] = jnp.sum(h2f * a2s_ref[...], axis=1)
        d_ref[0, 0] = jnp.sum(h2f * a2d_ref[...], axis=1)


def _tc2(acc3, den4, b1r, w2r, a2s, a2d):
    return pl.pallas_call(
        _tc2_body,
        grid=(NBLK, PLANES),
        in_specs=[
            pl.BlockSpec((1, BLK, PW), lambda i, p: (p, i, 0)),
            pl.BlockSpec((1, 1, 1, BLK), lambda i, p: (p // 2, i, 0, 0)),
            pl.BlockSpec((1, 1, PW), lambda i, p: (p, 0, 0)),
            pl.BlockSpec((1, PW, NCP), lambda i, p: (p, 0, 0)),
            pl.BlockSpec((1, NCP), lambda i, p: (0, 0)),
            pl.BlockSpec((1, NCP), lambda i, p: (0, 0)),
        ],
        out_specs=[
            pl.BlockSpec((BLK, NCP), lambda i, p: (i, 0)),
            pl.BlockSpec((1, 1, BLK), lambda i, p: (i, 0, 0)),
            pl.BlockSpec((1, 1, BLK), lambda i, p: (i, 0, 0)),
        ],
        out_shape=[
            jax.ShapeDtypeStruct((NP, NCP), _f32),
            jax.ShapeDtypeStruct((NBLK, 1, BLK), _f32),
            jax.ShapeDtypeStruct((NBLK, 1, BLK), _f32),
        ],
    )(acc3, den4, b1r, w2r, a2s, a2d)


def _tc3_body(acc_ref, den_ref, b2_ref, out_ref):
    s = acc_ref[0] + acc_ref[1]
    den = den_ref[0, 0, 0] + den_ref[1, 0, 0] + jnp.float32(1e-16)
    out_ref[...] = s[:, :NC] / den[:, None] + b2_ref[...]


def _tc3(acc2, den2, b2r):
    return pl.pallas_call(
        _tc3_body,
        grid=(NBLK,),
        in_specs=[
            pl.BlockSpec((2, BLK, NCP), lambda i: (0, i, 0)),
            pl.BlockSpec((2, 1, 1, BLK), lambda i: (0, i, 0, 0)),
            pl.BlockSpec((1, NC), lambda i: (0, 0)),
        ],
        out_specs=pl.BlockSpec((BLK, NC), lambda i: (i, 0)),
        out_shape=jax.ShapeDtypeStruct((N, NC), _f32),
    )(acc2, den2, b2r)


_make_sc1a = functools.cache(_make_sc1a)
_make_sc1b = functools.cache(_make_sc1b)
_make_sc2a = functools.cache(_make_sc2a)
_make_sc2b = functools.cache(_make_sc2b)


def kernel(x, edge_index, W1, att_src1, att_dst1, b1, W2, att_src2,
           att_dst2, b2):
    x_pad = jnp.pad(x, ((0, NP - N), (0, 0)))
    loops = jnp.arange(N, dtype=edge_index.dtype)
    src = jnp.concatenate([edge_index[0], loops]).astype(_i32)
    dst = jnp.concatenate([edge_index[1], loops]).astype(_i32)
    srcp = jnp.pad(src, (0, EP - ET))
    dstp = jnp.pad(dst, (0, EP - ET), constant_values=N)
    pk = jnp.bitwise_or(jnp.left_shift(srcp, PKB), dstp)

    ats = att_src1.reshape(PLANES, 1, PW)
    atd = att_dst1.reshape(PLANES, 1, PW)
    h1g, a_sT, a_dT = _tc1(x_pad, W1, ats, atd)
    h1flat = h1g.reshape(PLANES * NP, PW)
    a_s2 = a_sT.reshape(HEADS * NP)
    a_d2 = a_dT.reshape(HEADS * NP)

    w_all, den = _make_sc1a()(a_s2, a_d2, pk)
    acc = _make_sc1b()(h1flat, w_all, pk)
    acc3 = acc.reshape(PLANES, NP, PW)
    den4 = den.reshape(HEADS, NP).reshape(HEADS, NBLK, 1, BLK)

    b1r = b1.reshape(PLANES, 1, PW)
    w2r = jnp.pad(W2, ((0, 0), (0, NCP - NC))).reshape(PLANES, PW, NCP)
    a2sp = jnp.pad(att_src2, ((0, 0), (0, NCP - NC)))
    a2dp = jnp.pad(att_dst2, ((0, 0), (0, NCP - NC)))
    h2pad, a2s, a2d = _tc2(acc3, den4, b1r, w2r, a2sp, a2dp)

    a2sr = a2s.reshape(NP)
    a2dr = a2d.reshape(NP)
    w2_all, den2 = _make_sc2a()(a2sr, a2dr, pk)
    acc2 = _make_sc2b()(h2pad, w2_all, pk)
    out = _tc3(acc2.reshape(2, NP, NCP),
               den2.reshape(2, NP).reshape(2, NBLK, 1, BLK),
               b2.reshape(1, NC))
    return out


# speedup vs baseline: 11.5962x; 1.0836x over previous
import functools

import jax
import jax.numpy as jnp
from jax import lax
from jax.experimental import pallas as pl
from jax.experimental.pallas import tpu as pltpu
from jax.experimental.pallas import tpu_sc as plsc

N = 10000
NP = 10240
NBLK = 8
BLK = 1280
D_IN = 256
HEADS = 8
NC = 40
NCP = 128
PLANES = 16
PW = 128
E = 160000
ET = E + N
EP = 172032
KB = 64
CT1 = EP // 16
B1 = CT1 // KB
CT2 = EP // 2 // 16
B2 = CT2 // KB
DR = NP // 128
DRT = DR // 16
PKB = 14
PKM = (1 << PKB) - 1

_f32 = jnp.float32
_i32 = jnp.int32


def _mesh():
    return plsc.VectorSubcoreMesh(core_axis_name="c", subcore_axis_name="s",
                                  num_cores=2, num_subcores=16)


_SC_CP = functools.partial(pltpu.CompilerParams, needs_layout_passes=False)


def _tc1_body(x_ref, w_ref, ats_ref, atd_ref, h_ref, as_ref, ad_ref):
    p = pl.program_id(1)
    h = jnp.dot(x_ref[...], w_ref[...], preferred_element_type=_f32)
    h_ref[0] = h
    asp = jnp.sum(h * ats_ref[0], axis=1)
    adp = jnp.sum(h * atd_ref[0], axis=1)

    @pl.when(p % 2 == 0)
    def _():
        as_ref[0, 0, 0] = asp
        ad_ref[0, 0, 0] = adp

    @pl.when(p % 2 == 1)
    def _():
        as_ref[0, 0, 0] += asp
        ad_ref[0, 0, 0] += adp


def _tc1(x_pad, W1, ats, atd):
    return pl.pallas_call(
        _tc1_body,
        grid=(NBLK, PLANES),
        in_specs=[
            pl.BlockSpec((BLK, D_IN), lambda i, p: (i, 0)),
            pl.BlockSpec((D_IN, PW), lambda i, p: (0, p)),
            pl.BlockSpec((1, 1, PW), lambda i, p: (p, 0, 0)),
            pl.BlockSpec((1, 1, PW), lambda i, p: (p, 0, 0)),
        ],
        out_specs=[
            pl.BlockSpec((1, BLK, PW), lambda i, p: (p, i, 0)),
            pl.BlockSpec((1, 1, 1, BLK), lambda i, p: (p // 2, i, 0, 0)),
            pl.BlockSpec((1, 1, 1, BLK), lambda i, p: (p // 2, i, 0, 0)),
        ],
        out_shape=[
            jax.ShapeDtypeStruct((PLANES, NP, PW), _f32),
            jax.ShapeDtypeStruct((HEADS, NBLK, 1, BLK), _f32),
            jax.ShapeDtypeStruct((HEADS, NBLK, 1, BLK), _f32),
        ],
    )(x_pad, W1, ats, atd)


def _zero_vmem_rows(ref, nrows, width):
    z16 = jnp.zeros((16,), _f32)

    @pl.loop(0, nrows)
    def _(r):
        for j in range(width // 16):
            ref[r, pl.ds(16 * j, 16)] = z16


def _build_den_idx(den_idx):
    lane = jnp.arange(16, dtype=_i32)
    for g in range(DR // 16):
        den_idx[0, pl.ds(16 * g, 16)] = 16 * g + lane


def _edge_weights(pk_t, a_s, a_d, w_t, den_vm, ngroups):
    @pl.loop(0, ngroups)
    def _(g):
        pk16 = pk_t[pl.ds(16 * g, 16)]
        s16 = lax.shift_right_logical(pk16, PKB)
        d16 = jnp.bitwise_and(pk16, PKM)
        av = plsc.load_gather(a_s, [s16]) + plsc.load_gather(a_d, [d16])
        av = jnp.where(av >= 0.0, av, av * jnp.float32(0.2))
        w = jnp.exp(av)
        w_t[pl.ds(16 * g, 16)] = w
        row = lax.shift_right_logical(d16, 7)
        col = jnp.bitwise_and(d16, 127)
        plsc.addupdate_scatter(den_vm, [row, col], w)


def _scores_body(as_hbm, ad_hbm, pk_hbm, w_hbm, den_hbm,
                 pk_t, a_s, a_d, den_vm, w_tile, zden, den_idx, bnc, den_sh,
                 *, nheads, edges_per_tile, den_rows_out):
    cid = lax.axis_index("c")
    sid = lax.axis_index("s")

    if nheads == 1:
        e0 = cid * (EP // 2) + sid * edges_per_tile
    else:
        e0 = sid * edges_per_tile
    pltpu.sync_copy(pk_hbm.at[pl.ds(e0, edges_per_tile)], pk_t)
    _zero_vmem_rows(zden, 8, 128)
    _build_den_idx(den_idx)

    @pl.loop(0, nheads)
    def _(hh):
        head = cid * 4 + hh if nheads > 1 else jnp.int32(0)
        pltpu.sync_copy(as_hbm.at[pl.ds(head * NP, NP)], a_s)
        pltpu.sync_copy(ad_hbm.at[pl.ds(head * NP, NP)], a_d)
        _zero_vmem_rows(den_vm, DR, 128)

        @pl.when(sid < 10)
        def _():
            pltpu.sync_copy(zden, den_sh.at[pl.ds(sid * 8, 8)])

        plsc.subcore_barrier()
        _edge_weights(pk_t, a_s, a_d, w_tile, den_vm, edges_per_tile // 16)
        pltpu.sync_copy(den_vm, den_sh.at[den_idx.at[0]], add=True)
        plsc.subcore_barrier()

        @pl.when(sid < 10)
        def _():
            pltpu.sync_copy(den_sh.at[pl.ds(sid * 8, 8)], bnc)
            if nheads == 1:
                pltpu.sync_copy(
                    bnc, den_hbm.at[pl.ds(cid * DR + sid * 8, 8)])
            else:
                pltpu.sync_copy(bnc, den_hbm.at[head, pl.ds(sid * 8, 8)])

        if nheads == 1:
            pltpu.sync_copy(w_tile, w_hbm.at[pl.ds(e0, edges_per_tile)])
        else:
            pltpu.sync_copy(
                w_tile,
                w_hbm.at[pl.ds(head * EP + sid * edges_per_tile,
                               edges_per_tile)])


def _make_sc1a():
    body = functools.partial(_scores_body, nheads=4,
                             edges_per_tile=CT1, den_rows_out=DR)
    return pl.kernel(
        body,
        out_type=[
            jax.ShapeDtypeStruct((HEADS * EP,), _f32),
            jax.ShapeDtypeStruct((HEADS, DR, 128), _f32),
        ],
        mesh=_mesh(),
        compiler_params=_SC_CP(),
        scratch_types=[
            pltpu.VMEM((CT1,), _i32),
            pltpu.VMEM((NP,), _f32),
            pltpu.VMEM((NP,), _f32),
            pltpu.VMEM((DR, 128), _f32),
            pltpu.VMEM((CT1,), _f32),
            pltpu.VMEM((8, 128), _f32),
            pltpu.VMEM((1, DR), _i32),
            pltpu.VMEM((8, 128), _f32),
            pltpu.VMEM_SHARED((DR, 128), _f32),
        ],
    )


def _make_sc2a():
    body = functools.partial(_scores_body, nheads=1,
                             edges_per_tile=CT2, den_rows_out=2 * DR)
    return pl.kernel(
        body,
        out_type=[
            jax.ShapeDtypeStruct((EP,), _f32),
            jax.ShapeDtypeStruct((2 * DR, 128), _f32),
        ],
        mesh=_mesh(),
        compiler_params=_SC_CP(),
        scratch_types=[
            pltpu.VMEM((CT2,), _i32),
            pltpu.VMEM((NP,), _f32),
            pltpu.VMEM((NP,), _f32),
            pltpu.VMEM((DR, 128), _f32),
            pltpu.VMEM((CT2,), _f32),
            pltpu.VMEM((8, 128), _f32),
            pltpu.VMEM((1, DR), _i32),
            pltpu.VMEM((8, 128), _f32),
            pltpu.VMEM_SHARED((DR, 128), _f32),
        ],
    )


def _agg_loop(pk_t, w_t, idxg, dstb, rows, acc_sh, h_hbm, sems,
              pbase, nbatches, width):
    gsem0, gsem1, ssem0, ssem1 = sems
    gsems = (gsem0, gsem1)
    ssems = (ssem0, ssem1)

    def build(pi, b):
        base = b * KB
        for j in range(KB // 16):
            pk16 = pk_t[pl.ds(base + 16 * j, 16)]
            idxg[pi, pl.ds(16 * j, 16)] = (
                lax.shift_right_logical(pk16, PKB) + pbase)
            dstb[pi, pl.ds(16 * j, 16)] = jnp.bitwise_and(pk16, PKM)

    def scale(pi, b):
        base = b * KB

        @plsc.parallel_loop(0, KB, 1, unroll=4)
        def _(e):
            wv = plsc.load_gather(w_t, [jnp.full((16,), base + e, _i32)])
            for j in range(width // 16):
                rows[pi, e, pl.ds(16 * j, 16)] = (
                    rows[pi, e, pl.ds(16 * j, 16)] * wv)

    build(0, jnp.int32(0))
    pltpu.async_copy(h_hbm.at[idxg.at[0]], rows.at[0], gsems[0])

    @pl.loop(0, nbatches // 2)
    def _(bb):
        for par in range(2):
            b = 2 * bb + par
            cur, oth = par, 1 - par

            @pl.when(b > 0)
            def _():
                pltpu.make_async_copy(
                    rows.at[oth], acc_sh.at[dstb.at[oth]],
                    ssems[oth]).wait()

            @pl.when(b + 1 < nbatches)
            def _():
                build(oth, b + 1)
                pltpu.async_copy(h_hbm.at[idxg.at[oth]], rows.at[oth],
                                 gsems[oth])

            pltpu.make_async_copy(h_hbm.at[idxg.at[cur]], rows.at[cur],
                                  gsems[cur]).wait()
            scale(cur, b)
            pltpu.async_copy(rows.at[cur], acc_sh.at[dstb.at[cur]],
                             ssems[cur], add=True)

    last = (nbatches - 1) % 2
    pltpu.make_async_copy(rows.at[last], acc_sh.at[dstb.at[last]],
                          ssems[last]).wait()


def _sc1b_body(h1_hbm, w_hbm, pk_hbm, acc_hbm,
               pk_t, w_t, idxg, dstb, rows, zrow, acc_sh,
               gsem0, gsem1, ssem0, ssem1):
    cid = lax.axis_index("c")
    sid = lax.axis_index("s")
    sems = (gsem0, gsem1, ssem0, ssem1)

    pltpu.sync_copy(pk_hbm.at[pl.ds(sid * CT1, CT1)], pk_t)
    _zero_vmem_rows(zrow, 16, PW)

    @pl.loop(0, 4)
    def _(hh):
        head = cid * 4 + hh
        pltpu.sync_copy(w_hbm.at[pl.ds(head * EP + sid * CT1, CT1)], w_t)

        for half in range(2):
            pbase = (head * 2 + half) * NP

            @pl.loop(0, 40)
            def _(k):
                pltpu.sync_copy(zrow,
                                acc_sh.at[pl.ds(sid * 640 + 16 * k, 16)])

            plsc.subcore_barrier()
            _agg_loop(pk_t, w_t, idxg, dstb, rows, acc_sh, h1_hbm, sems,
                      pbase, B1, PW)
            plsc.subcore_barrier()

            @pl.loop(0, 10)
            def _(k):
                r0 = sid * 640 + 64 * k
                pltpu.sync_copy(acc_sh.at[pl.ds(r0, 64)], rows.at[0])
                pltpu.sync_copy(rows.at[0],
                                acc_hbm.at[pl.ds(pbase + r0, 64)])

            plsc.subcore_barrier()


def _make_sc1b():
    return pl.kernel(
        _sc1b_body,
        out_type=jax.ShapeDtypeStruct((PLANES * NP, PW), _f32),
        mesh=_mesh(),
        compiler_params=_SC_CP(),
        scratch_types=[
            pltpu.VMEM((CT1,), _i32),
            pltpu.VMEM((CT1,), _f32),
            pltpu.VMEM((2, KB), _i32),
            pltpu.VMEM((2, KB), _i32),
            pltpu.VMEM((2, KB, PW), _f32),
            pltpu.VMEM((16, PW), _f32),
            pltpu.VMEM_SHARED((NP, PW), _f32),
            pltpu.SemaphoreType.DMA,
            pltpu.SemaphoreType.DMA,
            pltpu.SemaphoreType.DMA,
            pltpu.SemaphoreType.DMA,
        ],
    )


def _sc2b_body(h2_hbm, w_hbm, pk_hbm, acc_hbm,
               pk_t, w_t, idxg, dstb, rows, zrow, acc_sh,
               gsem0, gsem1, ssem0, ssem1):
    cid = lax.axis_index("c")
    sid = lax.axis_index("s")
    sems = (gsem0, gsem1, ssem0, ssem1)

    e0 = cid * (EP // 2) + sid * CT2
    pltpu.sync_copy(pk_hbm.at[pl.ds(e0, CT2)], pk_t)
    pltpu.sync_copy(w_hbm.at[pl.ds(e0, CT2)], w_t)
    _zero_vmem_rows(zrow, 16, NCP)

    @pl.loop(0, 40)
    def _(k):
        pltpu.sync_copy(zrow, acc_sh.at[pl.ds(sid * 640 + 16 * k, 16)])

    plsc.subcore_barrier()
    _agg_loop(pk_t, w_t, idxg, dstb, rows, acc_sh, h2_hbm, sems,
              jnp.int32(0), B2, NCP)
    plsc.subcore_barrier()

    @pl.loop(0, 10)
    def _(k):
        r0 = sid * 640 + 64 * k
        pltpu.sync_copy(acc_sh.at[pl.ds(r0, 64)], rows.at[0])
        pltpu.sync_copy(rows.at[0], acc_hbm.at[pl.ds(cid * NP + r0, 64)])


def _make_sc2b():
    return pl.kernel(
        _sc2b_body,
        out_type=jax.ShapeDtypeStruct((2 * NP, NCP), _f32),
        mesh=_mesh(),
        compiler_params=_SC_CP(),
        scratch_types=[
            pltpu.VMEM((CT2,), _i32),
            pltpu.VMEM((CT2,), _f32),
            pltpu.VMEM((2, KB), _i32),
            pltpu.VMEM((2, KB), _i32),
            pltpu.VMEM((2, KB, NCP), _f32),
            pltpu.VMEM((16, NCP), _f32),
            pltpu.VMEM_SHARED((NP, NCP), _f32),
            pltpu.SemaphoreType.DMA,
            pltpu.SemaphoreType.DMA,
            pltpu.SemaphoreType.DMA,
            pltpu.SemaphoreType.DMA,
        ],
    )


def _tc2_body(acc_ref, den_ref, b1_ref, w2_ref, a2s_ref, a2d_ref,
              h2_ref, s_ref, d_ref):
    p = pl.program_id(1)
    den = den_ref[0, 0, 0] + jnp.float32(1e-16)
    x2 = jnp.maximum(acc_ref[0] / den[:, None] + b1_ref[0], 0.0)
    hp = jnp.dot(x2, w2_ref[0], preferred_element_type=_f32)

    @pl.when(p == 0)
    def _():
        h2_ref[...] = hp

    @pl.when(p > 0)
    def _():
        h2_ref[...] += hp

    @pl.when(p == PLANES - 1)
    def _():
        h2f = h2_ref[...]
        s_ref[0, 0] = jnp.sum(h2f * a2s_ref[...], axis=1)
        d_ref[0, 0] = jnp.sum(h2f * a2d_ref[...], axis=1)


def _tc2(acc3, den4, b1r, w2r, a2s, a2d):
    return pl.pallas_call(
        _tc2_body,
        grid=(NBLK, PLANES),
        in_specs=[
            pl.BlockSpec((1, BLK, PW), lambda i, p: (p, i, 0)),
            pl.BlockSpec((1, 1, 1, BLK), lambda i, p: (p // 2, i, 0, 0)),
            pl.BlockSpec((1, 1, PW), lambda i, p: (p, 0, 0)),
            pl.BlockSpec((1, PW, NCP), lambda i, p: (p, 0, 0)),
            pl.BlockSpec((1, NCP), lambda i, p: (0, 0)),
            pl.BlockSpec((1, NCP), lambda i, p: (0, 0)),
        ],
        out_specs=[
            pl.BlockSpec((BLK, NCP), lambda i, p: (i, 0)),
            pl.BlockSpec((1, 1, BLK), lambda i, p: (i, 0, 0)),
            pl.BlockSpec((1, 1, BLK), lambda i, p: (i, 0, 0)),
        ],
        out_shape=[
            jax.ShapeDtypeStruct((NP, NCP), _f32),
            jax.ShapeDtypeStruct((NBLK, 1, BLK), _f32),
            jax.ShapeDtypeStruct((NBLK, 1, BLK), _f32),
        ],
    )(acc3, den4, b1r, w2r, a2s, a2d)


def _tc3_body(acc_ref, den_ref, b2_ref, out_ref):
    s = acc_ref[0] + acc_ref[1]
    den = den_ref[0, 0, 0] + den_ref[1, 0, 0] + jnp.float32(1e-16)
    out_ref[...] = s[:, :NC] / den[:, None] + b2_ref[...]


def _tc3(acc2, den2, b2r):
    return pl.pallas_call(
        _tc3_body,
        grid=(NBLK,),
        in_specs=[
            pl.BlockSpec((2, BLK, NCP), lambda i: (0, i, 0)),
            pl.BlockSpec((2, 1, 1, BLK), lambda i: (0, i, 0, 0)),
            pl.BlockSpec((1, NC), lambda i: (0, 0)),
        ],
        out_specs=pl.BlockSpec((BLK, NC), lambda i: (i, 0)),
        out_shape=jax.ShapeDtypeStruct((N, NC), _f32),
    )(acc2, den2, b2r)


_make_sc1a = functools.cache(_make_sc1a)
_make_sc1b = functools.cache(_make_sc1b)
_make_sc2a = functools.cache(_make_sc2a)
_make_sc2b = functools.cache(_make_sc2b)


def kernel(x, edge_index, W1, att_src1, att_dst1, b1, W2, att_src2,
           att_dst2, b2):
    x_pad = jnp.pad(x, ((0, NP - N), (0, 0)))
    loops = jnp.arange(N, dtype=edge_index.dtype)
    src = jnp.concatenate([edge_index[0], loops]).astype(_i32)
    dst = jnp.concatenate([edge_index[1], loops]).astype(_i32)
    srcp = jnp.pad(src, (0, EP - ET))
    dstp = jnp.pad(dst, (0, EP - ET), constant_values=N)
    pk = jnp.bitwise_or(jnp.left_shift(srcp, PKB), dstp)

    ats = att_src1.reshape(PLANES, 1, PW)
    atd = att_dst1.reshape(PLANES, 1, PW)
    h1g, a_sT, a_dT = _tc1(x_pad, W1, ats, atd)
    h1flat = h1g.reshape(PLANES * NP, PW)
    a_s2 = a_sT.reshape(HEADS * NP)
    a_d2 = a_dT.reshape(HEADS * NP)

    w_all, den = _make_sc1a()(a_s2, a_d2, pk)
    acc = _make_sc1b()(h1flat, w_all, pk)
    acc3 = acc.reshape(PLANES, NP, PW)
    den4 = den.reshape(HEADS, NP).reshape(HEADS, NBLK, 1, BLK)

    b1r = b1.reshape(PLANES, 1, PW)
    w2r = jnp.pad(W2, ((0, 0), (0, NCP - NC))).reshape(PLANES, PW, NCP)
    a2sp = jnp.pad(att_src2, ((0, 0), (0, NCP - NC)))
    a2dp = jnp.pad(att_dst2, ((0, 0), (0, NCP - NC)))
    h2pad, a2s, a2d = _tc2(acc3, den4, b1r, w2r, a2sp, a2dp)

    a2sr = a2s.reshape(NP)
    a2dr = a2d.reshape(NP)
    w2_all, den2 = _make_sc2a()(a2sr, a2dr, pk)
    acc2 = _make_sc2b()(h2pad, w2_all, pk)
    out = _tc3(acc2.reshape(2, NP, NCP),
               den2.reshape(2, NP).reshape(2, NBLK, 1, BLK),
               b2.reshape(1, NC))
    return out


# speedup vs baseline: 11.6114x; 1.0013x over previous
import functools

import jax
import jax.numpy as jnp
from jax import lax
from jax.experimental import pallas as pl
from jax.experimental.pallas import tpu as pltpu
from jax.experimental.pallas import tpu_sc as plsc

N = 10000
NP = 10240
NBLK = 8
BLK = 1280
D_IN = 256
HEADS = 8
NC = 40
NCP = 128
PLANES = 16
PW = 128
E = 160000
ET = E + N
EP = 172032
KB = 64
CT1 = EP // 16
B1 = CT1 // KB
CT2 = EP // 2 // 16
B2 = CT2 // KB
DR = NP // 128
DRT = DR // 16
PKB = 14
PKM = (1 << PKB) - 1

_f32 = jnp.float32
_i32 = jnp.int32


def _mesh():
    return plsc.VectorSubcoreMesh(core_axis_name="c", subcore_axis_name="s",
                                  num_cores=2, num_subcores=16)


_SC_CP = functools.partial(pltpu.CompilerParams, needs_layout_passes=False)


def _tc1_body(x_ref, w_ref, ats_ref, atd_ref, h_ref, as_ref, ad_ref):
    p = pl.program_id(1)
    h = jnp.dot(x_ref[...], w_ref[...], preferred_element_type=_f32)
    h_ref[0] = h
    asp = jnp.sum(h * ats_ref[0], axis=1)
    adp = jnp.sum(h * atd_ref[0], axis=1)

    @pl.when(p % 2 == 0)
    def _():
        as_ref[0, 0, 0] = asp
        ad_ref[0, 0, 0] = adp

    @pl.when(p % 2 == 1)
    def _():
        as_ref[0, 0, 0] += asp
        ad_ref[0, 0, 0] += adp


def _tc1(x_pad, W1, ats, atd):
    return pl.pallas_call(
        _tc1_body,
        grid=(NBLK, PLANES),
        in_specs=[
            pl.BlockSpec((BLK, D_IN), lambda i, p: (i, 0)),
            pl.BlockSpec((D_IN, PW), lambda i, p: (0, p)),
            pl.BlockSpec((1, 1, PW), lambda i, p: (p, 0, 0)),
            pl.BlockSpec((1, 1, PW), lambda i, p: (p, 0, 0)),
        ],
        out_specs=[
            pl.BlockSpec((1, BLK, PW), lambda i, p: (p, i, 0)),
            pl.BlockSpec((1, 1, 1, BLK), lambda i, p: (p // 2, i, 0, 0)),
            pl.BlockSpec((1, 1, 1, BLK), lambda i, p: (p // 2, i, 0, 0)),
        ],
        out_shape=[
            jax.ShapeDtypeStruct((PLANES, NP, PW), _f32),
            jax.ShapeDtypeStruct((HEADS, NBLK, 1, BLK), _f32),
            jax.ShapeDtypeStruct((HEADS, NBLK, 1, BLK), _f32),
        ],
    )(x_pad, W1, ats, atd)


def _zero_vmem_rows(ref, nrows, width):
    z16 = jnp.zeros((16,), _f32)

    @pl.loop(0, nrows)
    def _(r):
        for j in range(width // 16):
            ref[r, pl.ds(16 * j, 16)] = z16


def _build_den_idx(den_idx):
    lane = jnp.arange(16, dtype=_i32)
    for g in range(DR // 16):
        den_idx[0, pl.ds(16 * g, 16)] = 16 * g + lane


def _edge_weights(pk_t, a_s, a_d, w_t, den_vm, ngroups):
    @pl.loop(0, ngroups)
    def _(g):
        pk16 = pk_t[pl.ds(16 * g, 16)]
        s16 = lax.shift_right_logical(pk16, PKB)
        d16 = jnp.bitwise_and(pk16, PKM)
        av = plsc.load_gather(a_s, [s16]) + plsc.load_gather(a_d, [d16])
        av = jnp.where(av >= 0.0, av, av * jnp.float32(0.2))
        w = jnp.exp(av)
        w_t[pl.ds(16 * g, 16)] = w
        row = lax.shift_right_logical(d16, 7)
        col = jnp.bitwise_and(d16, 127)
        plsc.addupdate_scatter(den_vm, [row, col], w)


def _scores_body(as_hbm, ad_hbm, pk_hbm, w_hbm, den_hbm,
                 pk_t, a_s, a_d, den_vm, w_tile, zden, den_idx, bnc, den_sh,
                 *, nheads, edges_per_tile, den_rows_out):
    cid = lax.axis_index("c")
    sid = lax.axis_index("s")

    if nheads == 1:
        e0 = cid * (EP // 2) + sid * edges_per_tile
    else:
        e0 = sid * edges_per_tile
    pltpu.sync_copy(pk_hbm.at[pl.ds(e0, edges_per_tile)], pk_t)
    _zero_vmem_rows(zden, 8, 128)
    _build_den_idx(den_idx)

    @pl.loop(0, nheads)
    def _(hh):
        head = cid * 4 + hh if nheads > 1 else jnp.int32(0)
        pltpu.sync_copy(as_hbm.at[pl.ds(head * NP, NP)], a_s)
        pltpu.sync_copy(ad_hbm.at[pl.ds(head * NP, NP)], a_d)
        _zero_vmem_rows(den_vm, DR, 128)

        @pl.when(sid < 10)
        def _():
            pltpu.sync_copy(zden, den_sh.at[pl.ds(sid * 8, 8)])

        plsc.subcore_barrier()
        _edge_weights(pk_t, a_s, a_d, w_tile, den_vm, edges_per_tile // 16)
        pltpu.sync_copy(den_vm, den_sh.at[den_idx.at[0]], add=True)
        plsc.subcore_barrier()

        @pl.when(sid < 10)
        def _():
            pltpu.sync_copy(den_sh.at[pl.ds(sid * 8, 8)], bnc)
            if nheads == 1:
                pltpu.sync_copy(
                    bnc, den_hbm.at[pl.ds(cid * DR + sid * 8, 8)])
            else:
                pltpu.sync_copy(bnc, den_hbm.at[head, pl.ds(sid * 8, 8)])

        if nheads == 1:
            pltpu.sync_copy(w_tile, w_hbm.at[pl.ds(e0, edges_per_tile)])
        else:
            pltpu.sync_copy(
                w_tile,
                w_hbm.at[pl.ds(head * EP + sid * edges_per_tile,
                               edges_per_tile)])


def _make_sc1a():
    body = functools.partial(_scores_body, nheads=4,
                             edges_per_tile=CT1, den_rows_out=DR)
    return pl.kernel(
        body,
        out_type=[
            jax.ShapeDtypeStruct((HEADS * EP,), _f32),
            jax.ShapeDtypeStruct((HEADS, DR, 128), _f32),
        ],
        mesh=_mesh(),
        compiler_params=_SC_CP(),
        scratch_types=[
            pltpu.VMEM((CT1,), _i32),
            pltpu.VMEM((NP,), _f32),
            pltpu.VMEM((NP,), _f32),
            pltpu.VMEM((DR, 128), _f32),
            pltpu.VMEM((CT1,), _f32),
            pltpu.VMEM((8, 128), _f32),
            pltpu.VMEM((1, DR), _i32),
            pltpu.VMEM((8, 128), _f32),
            pltpu.VMEM_SHARED((DR, 128), _f32),
        ],
    )


def _make_sc2a():
    body = functools.partial(_scores_body, nheads=1,
                             edges_per_tile=CT2, den_rows_out=2 * DR)
    return pl.kernel(
        body,
        out_type=[
            jax.ShapeDtypeStruct((EP,), _f32),
            jax.ShapeDtypeStruct((2 * DR, 128), _f32),
        ],
        mesh=_mesh(),
        compiler_params=_SC_CP(),
        scratch_types=[
            pltpu.VMEM((CT2,), _i32),
            pltpu.VMEM((NP,), _f32),
            pltpu.VMEM((NP,), _f32),
            pltpu.VMEM((DR, 128), _f32),
            pltpu.VMEM((CT2,), _f32),
            pltpu.VMEM((8, 128), _f32),
            pltpu.VMEM((1, DR), _i32),
            pltpu.VMEM((8, 128), _f32),
            pltpu.VMEM_SHARED((DR, 128), _f32),
        ],
    )


def _agg_loop(pk_t, w_t, idxg, dstb, rows, acc_sh, h_hbm, sems,
              pbase, nbatches, width):
    gsem0, gsem1, ssem0, ssem1 = sems
    gsems = (gsem0, gsem1)
    ssems = (ssem0, ssem1)

    def build(pi, b):
        base = b * KB
        for j in range(KB // 16):
            pk16 = pk_t[pl.ds(base + 16 * j, 16)]
            idxg[pi, pl.ds(16 * j, 16)] = (
                lax.shift_right_logical(pk16, PKB) + pbase)
            dstb[pi, pl.ds(16 * j, 16)] = jnp.bitwise_and(pk16, PKM)

    def scale(pi, b):
        base = b * KB

        @plsc.parallel_loop(0, KB, 1, unroll=8)
        def _(e):
            wv = plsc.load_gather(w_t, [jnp.full((16,), base + e, _i32)])
            for j in range(width // 16):
                rows[pi, e, pl.ds(16 * j, 16)] = (
                    rows[pi, e, pl.ds(16 * j, 16)] * wv)

    build(0, jnp.int32(0))
    pltpu.async_copy(h_hbm.at[idxg.at[0]], rows.at[0], gsems[0])

    @pl.loop(0, nbatches // 2)
    def _(bb):
        for par in range(2):
            b = 2 * bb + par
            cur, oth = par, 1 - par

            @pl.when(b > 0)
            def _():
                pltpu.make_async_copy(
                    rows.at[oth], acc_sh.at[dstb.at[oth]],
                    ssems[oth]).wait()

            @pl.when(b + 1 < nbatches)
            def _():
                build(oth, b + 1)
                pltpu.async_copy(h_hbm.at[idxg.at[oth]], rows.at[oth],
                                 gsems[oth])

            pltpu.make_async_copy(h_hbm.at[idxg.at[cur]], rows.at[cur],
                                  gsems[cur]).wait()
            scale(cur, b)
            pltpu.async_copy(rows.at[cur], acc_sh.at[dstb.at[cur]],
                             ssems[cur], add=True)

    last = (nbatches - 1) % 2
    pltpu.make_async_copy(rows.at[last], acc_sh.at[dstb.at[last]],
                          ssems[last]).wait()


def _sc1b_body(h1_hbm, w_hbm, pk_hbm, acc_hbm,
               pk_t, w_t, idxg, dstb, rows, zrow, acc_sh,
               gsem0, gsem1, ssem0, ssem1):
    cid = lax.axis_index("c")
    sid = lax.axis_index("s")
    sems = (gsem0, gsem1, ssem0, ssem1)

    pltpu.sync_copy(pk_hbm.at[pl.ds(sid * CT1, CT1)], pk_t)
    _zero_vmem_rows(zrow, 16, PW)

    @pl.loop(0, 4)
    def _(hh):
        head = cid * 4 + hh
        pltpu.sync_copy(w_hbm.at[pl.ds(head * EP + sid * CT1, CT1)], w_t)

        for half in range(2):
            pbase = (head * 2 + half) * NP

            @pl.loop(0, 40)
            def _(k):
                pltpu.sync_copy(zrow,
                                acc_sh.at[pl.ds(sid * 640 + 16 * k, 16)])

            plsc.subcore_barrier()
            _agg_loop(pk_t, w_t, idxg, dstb, rows, acc_sh, h1_hbm, sems,
                      pbase, B1, PW)
            plsc.subcore_barrier()

            @pl.loop(0, 10)
            def _(k):
                r0 = sid * 640 + 64 * k
                pltpu.sync_copy(acc_sh.at[pl.ds(r0, 64)], rows.at[0])
                pltpu.sync_copy(rows.at[0],
                                acc_hbm.at[pl.ds(pbase + r0, 64)])

            plsc.subcore_barrier()


def _make_sc1b():
    return pl.kernel(
        _sc1b_body,
        out_type=jax.ShapeDtypeStruct((PLANES * NP, PW), _f32),
        mesh=_mesh(),
        compiler_params=_SC_CP(),
        scratch_types=[
            pltpu.VMEM((CT1,), _i32),
            pltpu.VMEM((CT1,), _f32),
            pltpu.VMEM((2, KB), _i32),
            pltpu.VMEM((2, KB), _i32),
            pltpu.VMEM((2, KB, PW), _f32),
            pltpu.VMEM((16, PW), _f32),
            pltpu.VMEM_SHARED((NP, PW), _f32),
            pltpu.SemaphoreType.DMA,
            pltpu.SemaphoreType.DMA,
            pltpu.SemaphoreType.DMA,
            pltpu.SemaphoreType.DMA,
        ],
    )


def _sc2b_body(h2_hbm, w_hbm, pk_hbm, acc_hbm,
               pk_t, w_t, idxg, dstb, rows, zrow, acc_sh,
               gsem0, gsem1, ssem0, ssem1):
    cid = lax.axis_index("c")
    sid = lax.axis_index("s")
    sems = (gsem0, gsem1, ssem0, ssem1)

    e0 = cid * (EP // 2) + sid * CT2
    pltpu.sync_copy(pk_hbm.at[pl.ds(e0, CT2)], pk_t)
    pltpu.sync_copy(w_hbm.at[pl.ds(e0, CT2)], w_t)
    _zero_vmem_rows(zrow, 16, NCP)

    @pl.loop(0, 40)
    def _(k):
        pltpu.sync_copy(zrow, acc_sh.at[pl.ds(sid * 640 + 16 * k, 16)])

    plsc.subcore_barrier()
    _agg_loop(pk_t, w_t, idxg, dstb, rows, acc_sh, h2_hbm, sems,
              jnp.int32(0), B2, NCP)
    plsc.subcore_barrier()

    @pl.loop(0, 10)
    def _(k):
        r0 = sid * 640 + 64 * k
        pltpu.sync_copy(acc_sh.at[pl.ds(r0, 64)], rows.at[0])
        pltpu.sync_copy(rows.at[0], acc_hbm.at[pl.ds(cid * NP + r0, 64)])


def _make_sc2b():
    return pl.kernel(
        _sc2b_body,
        out_type=jax.ShapeDtypeStruct((2 * NP, NCP), _f32),
        mesh=_mesh(),
        compiler_params=_SC_CP(),
        scratch_types=[
            pltpu.VMEM((CT2,), _i32),
            pltpu.VMEM((CT2,), _f32),
            pltpu.VMEM((2, KB), _i32),
            pltpu.VMEM((2, KB), _i32),
            pltpu.VMEM((2, KB, NCP), _f32),
            pltpu.VMEM((16, NCP), _f32),
            pltpu.VMEM_SHARED((NP, NCP), _f32),
            pltpu.SemaphoreType.DMA,
            pltpu.SemaphoreType.DMA,
            pltpu.SemaphoreType.DMA,
            pltpu.SemaphoreType.DMA,
        ],
    )


def _tc2_body(acc_ref, den_ref, b1_ref, w2_ref, a2s_ref, a2d_ref,
              h2_ref, s_ref, d_ref):
    p = pl.program_id(1)
    den = den_ref[0, 0, 0] + jnp.float32(1e-16)
    x2 = jnp.maximum(acc_ref[0] / den[:, None] + b1_ref[0], 0.0)
    hp = jnp.dot(x2, w2_ref[0], preferred_element_type=_f32)

    @pl.when(p == 0)
    def _():
        h2_ref[...] = hp

    @pl.when(p > 0)
    def _():
        h2_ref[...] += hp

    @pl.when(p == PLANES - 1)
    def _():
        h2f = h2_ref[...]
        s_ref[0, 0] = jnp.sum(h2f * a2s_ref[...], axis=1)
        d_ref[0, 0] = jnp.sum(h2f * a2d_ref[...], axis=1)


def _tc2(acc3, den4, b1r, w2r, a2s, a2d):
    return pl.pallas_call(
        _tc2_body,
        grid=(NBLK, PLANES),
        in_specs=[
            pl.BlockSpec((1, BLK, PW), lambda i, p: (p, i, 0)),
            pl.BlockSpec((1, 1, 1, BLK), lambda i, p: (p // 2, i, 0, 0)),
            pl.BlockSpec((1, 1, PW), lambda i, p: (p, 0, 0)),
            pl.BlockSpec((1, PW, NCP), lambda i, p: (p, 0, 0)),
            pl.BlockSpec((1, NCP), lambda i, p: (0, 0)),
            pl.BlockSpec((1, NCP), lambda i, p: (0, 0)),
        ],
        out_specs=[
            pl.BlockSpec((BLK, NCP), lambda i, p: (i, 0)),
            pl.BlockSpec((1, 1, BLK), lambda i, p: (i, 0, 0)),
            pl.BlockSpec((1, 1, BLK), lambda i, p: (i, 0, 0)),
        ],
        out_shape=[
            jax.ShapeDtypeStruct((NP, NCP), _f32),
            jax.ShapeDtypeStruct((NBLK, 1, BLK), _f32),
            jax.ShapeDtypeStruct((NBLK, 1, BLK), _f32),
        ],
    )(acc3, den4, b1r, w2r, a2s, a2d)


def _tc3_body(acc_ref, den_ref, b2_ref, out_ref):
    s = acc_ref[0] + acc_ref[1]
    den = den_ref[0, 0, 0] + den_ref[1, 0, 0] + jnp.float32(1e-16)
    out_ref[...] = s[:, :NC] / den[:, None] + b2_ref[...]


def _tc3(acc2, den2, b2r):
    return pl.pallas_call(
        _tc3_body,
        grid=(NBLK,),
        in_specs=[
            pl.BlockSpec((2, BLK, NCP), lambda i: (0, i, 0)),
            pl.BlockSpec((2, 1, 1, BLK), lambda i: (0, i, 0, 0)),
            pl.BlockSpec((1, NC), lambda i: (0, 0)),
        ],
        out_specs=pl.BlockSpec((BLK, NC), lambda i: (i, 0)),
        out_shape=jax.ShapeDtypeStruct((N, NC), _f32),
    )(acc2, den2, b2r)


_make_sc1a = functools.cache(_make_sc1a)
_make_sc1b = functools.cache(_make_sc1b)
_make_sc2a = functools.cache(_make_sc2a)
_make_sc2b = functools.cache(_make_sc2b)


def kernel(x, edge_index, W1, att_src1, att_dst1, b1, W2, att_src2,
           att_dst2, b2):
    x_pad = jnp.pad(x, ((0, NP - N), (0, 0)))
    loops = jnp.arange(N, dtype=edge_index.dtype)
    src = jnp.concatenate([edge_index[0], loops]).astype(_i32)
    dst = jnp.concatenate([edge_index[1], loops]).astype(_i32)
    srcp = jnp.pad(src, (0, EP - ET))
    dstp = jnp.pad(dst, (0, EP - ET), constant_values=N)
    pk = jnp.bitwise_or(jnp.left_shift(srcp, PKB), dstp)

    ats = att_src1.reshape(PLANES, 1, PW)
    atd = att_dst1.reshape(PLANES, 1, PW)
    h1g, a_sT, a_dT = _tc1(x_pad, W1, ats, atd)
    h1flat = h1g.reshape(PLANES * NP, PW)
    a_s2 = a_sT.reshape(HEADS * NP)
    a_d2 = a_dT.reshape(HEADS * NP)

    w_all, den = _make_sc1a()(a_s2, a_d2, pk)
    acc = _make_sc1b()(h1flat, w_all, pk)
    acc3 = acc.reshape(PLANES, NP, PW)
    den4 = den.reshape(HEADS, NP).reshape(HEADS, NBLK, 1, BLK)

    b1r = b1.reshape(PLANES, 1, PW)
    w2r = jnp.pad(W2, ((0, 0), (0, NCP - NC))).reshape(PLANES, PW, NCP)
    a2sp = jnp.pad(att_src2, ((0, 0), (0, NCP - NC)))
    a2dp = jnp.pad(att_dst2, ((0, 0), (0, NCP - NC)))
    h2pad, a2s, a2d = _tc2(acc3, den4, b1r, w2r, a2sp, a2dp)

    a2sr = a2s.reshape(NP)
    a2dr = a2d.reshape(NP)
    w2_all, den2 = _make_sc2a()(a2sr, a2dr, pk)
    acc2 = _make_sc2b()(h2pad, w2_all, pk)
    out = _tc3(acc2.reshape(2, NP, NCP),
               den2.reshape(2, NP).reshape(2, NBLK, 1, BLK),
               b2.reshape(1, NC))
    return out
